# baseline (speedup 1.0000x reference)
import jax
import jax.numpy as jnp
from jax import lax
from jax.experimental import pallas as pl
from jax.experimental.pallas import tpu as pltpu
from jax.experimental.pallas import tpu_sc as plsc

_B = 16384
_E = 64
_L = 16
_NC = 2
_NS = 16
_NW = _NC * _NS
_RPW = _B // _NW
_NBLK = _RPW // _L


def _router_body(x_hbm, idx_hbm, pmax_hbm, pmax2_hbm, part_hbm,
                 x_v, ebuf, colacc, cnt64, idxout, pmaxout, partv,
                 s_buf, m_buf, inv_buf, dma_sem):
    cid = lax.axis_index("c")
    sid = lax.axis_index("s")
    wid = sid * _NC + cid
    base = wid * _RPW

    copies = [
        pltpu.async_copy(x_hbm.at[c, pl.ds(4 * wid, 4)], x_v.at[c], dma_sem)
        for c in range(_E)
    ]
    for cp in copies:
        cp.wait()

    zf = jnp.zeros((_L,), jnp.float32)
    for c in range(_E):
        colacc[pl.ds(c * _L, _L)] = zf
    for j in range(_E // _L):
        cnt64[pl.ds(j * _L, _L)] = zf

    lane = lax.iota(jnp.int32, _L)
    ones = jnp.ones((_L,), jnp.float32)

    def block(b, carry):
        t = b >> 3
        q = (b & 7) * _L
        minf = jnp.full((_L,), -jnp.inf, jnp.float32)

        @plsc.parallel_loop(0, _E, 1, unroll=16,
                            carry=(zf, minf, jnp.zeros((_L,), jnp.int32)))
        def pass1(c, cr):
            s, m, idxv = cr
            v = x_v[c, t, pl.ds(q, _L)]
            e = jnp.exp(v)
            ebuf[pl.ds((b * _E + c) * _L, _L)] = e
            gt = v > m
            idxv = jnp.where(gt, c, idxv)
            m = jnp.maximum(m, v)
            return (s + e, m, idxv)

        s, m, idxv = pass1
        s_buf[pl.ds(b * _L, _L)] = s
        m_buf[pl.ds(b * _L, _L)] = m
        idxout[pl.ds(b * _L, _L)] = idxv
        plsc.addupdate_scatter(cnt64, [idxv], ones)
        return carry

    lax.fori_loop(0, _NBLK, block, 0)

    @plsc.parallel_loop(0, _NBLK, 1, unroll=4)
    def finalize(b):
        s = s_buf[pl.ds(b * _L, _L)]
        inv = 1.0 / s
        inv_buf[pl.ds(b * _L, _L)] = inv
        pmaxout[pl.ds(b * _L, _L)] = jnp.exp(m_buf[pl.ds(b * _L, _L)]) * inv

    def block2(b, carry):
        inv = inv_buf[pl.ds(b * _L, _L)]

        @plsc.parallel_loop(0, _E, 1, unroll=16)
        def pass2(c):
            q = ebuf[pl.ds((b * _E + c) * _L, _L)] * inv
            colacc[pl.ds(c * _L, _L)] = colacc[pl.ds(c * _L, _L)] + q

        return carry

    lax.fori_loop(0, _NBLK, block2, 0)

    for j in range(_E // _L):
        accv = zf
        for k in range(_L):
            s = jnp.sum(colacc[pl.ds((j * _L + k) * _L, _L)])
            accv = jnp.where(lane == k, jnp.broadcast_to(s, (_L,)), accv)
        partv[pl.ds(j * _L, _L)] = accv
    for j in range(_E // _L):
        partv[pl.ds(_E + j * _L, _L)] = cnt64[pl.ds(j * _L, _L)]

    pltpu.sync_copy(idxout, idx_hbm.at[pl.ds(base, _RPW)])
    pltpu.sync_copy(pmaxout, pmax_hbm.at[pl.ds(base, _RPW)])
    pltpu.sync_copy(pmaxout, pmax2_hbm.at[pl.ds(base, _RPW)])
    pltpu.sync_copy(partv, part_hbm.at[wid])


def _tc_finish_body(part_ref, out_ref):
    p = part_ref[...]
    tot = jnp.sum(p, axis=0, keepdims=True)
    inv_b = 1.0 / _B
    mean = tot[:, :_E] * inv_b
    frac = tot[:, _E:] * inv_b
    d = jnp.sum(frac * mean)
    loss = _E * d - 1.0
    fmin = jnp.min(frac)
    fmax = jnp.max(frac)
    fmean = jnp.sum(frac) * (1.0 / _E)
    var = jnp.sum((frac - fmean) ** 2) * (1.0 / (_E - 1))
    std = jnp.sqrt(var)
    sel = lax.broadcasted_iota(jnp.int32, (1, 8), 1)
    out = jnp.where(sel == 0, loss,
          jnp.where(sel == 1, fmin,
          jnp.where(sel == 2, fmax, std)))
    out_ref[...] = out


def kernel(route_logits):
    mesh = plsc.VectorSubcoreMesh(core_axis_name="c", subcore_axis_name="s")
    router = pl.kernel(
        _router_body,
        out_type=(
            jax.ShapeDtypeStruct((_B,), jnp.int32),
            jax.ShapeDtypeStruct((_B,), jnp.float32),
            jax.ShapeDtypeStruct((_B,), jnp.float32),
            jax.ShapeDtypeStruct((_NW, 2 * _E), jnp.float32),
        ),
        mesh=mesh,
        compiler_params=pltpu.CompilerParams(needs_layout_passes=False),
        scratch_types=[
            pltpu.VMEM((_E, 4, 128), jnp.float32),
            pltpu.VMEM((_RPW * _E,), jnp.float32),
            pltpu.VMEM((_E * _L,), jnp.float32),
            pltpu.VMEM((_E,), jnp.float32),
            pltpu.VMEM((_RPW,), jnp.int32),
            pltpu.VMEM((_RPW,), jnp.float32),
            pltpu.VMEM((2 * _E,), jnp.float32),
            pltpu.VMEM((_RPW,), jnp.float32),
            pltpu.VMEM((_RPW,), jnp.float32),
            pltpu.VMEM((_RPW,), jnp.float32),
            pltpu.SemaphoreType.DMA,
        ],
    )
    finish = pl.pallas_call(
        _tc_finish_body,
        out_shape=jax.ShapeDtypeStruct((1, 8), jnp.float32),
    )
    xt = route_logits.T.reshape(_E, 128, 128)
    route_idx, route_prob_max, route_mult, partials = router(xt)
    scal = finish(partials)
    return (route_idx, route_mult, scal[0, 0], route_prob_max,
            scal[0, 1], scal[0, 2], scal[0, 3])


# speedup vs baseline: 1.0493x; 1.0493x over previous
import jax
import jax.numpy as jnp
from jax import lax
from jax.experimental import pallas as pl
from jax.experimental.pallas import tpu as pltpu
from jax.experimental.pallas import tpu_sc as plsc

_B = 16384
_E = 64
_L = 16
_NC = 2
_NS = 16
_NW = _NC * _NS
_RPW = _B // _NW
_NBLK = _RPW // _L


def _router_body(x_hbm, idx_hbm, pmax_hbm, pmax2_hbm, part_hbm,
                 x_v, ebuf, colacc, cnt64, idxout, pmaxout, partv,
                 s_buf, m_buf, inv_buf, dma_sem):
    cid = lax.axis_index("c")
    sid = lax.axis_index("s")
    wid = sid * _NC + cid
    base = wid * _RPW

    copies = [
        pltpu.async_copy(x_hbm.at[c, pl.ds(4 * wid, 4)], x_v.at[c], dma_sem)
        for c in range(_E)
    ]
    for cp in copies:
        cp.wait()

    zf = jnp.zeros((_L,), jnp.float32)
    for c in range(_E):
        colacc[pl.ds(c * _L, _L)] = zf
    for j in range(_E // _L):
        cnt64[pl.ds(j * _L, _L)] = zf

    lane = lax.iota(jnp.int32, _L)
    ones = jnp.ones((_L,), jnp.float32)

    def block(b, carry):
        t = b >> 3
        q = (b & 7) * _L
        minf = jnp.full((_L,), -jnp.inf, jnp.float32)

        @plsc.parallel_loop(0, _E, 1, unroll=8,
                            carry=(zf, minf, jnp.zeros((_L,), jnp.int32)))
        def pass1(c, cr):
            s, m, idxv = cr
            v = x_v[c, t, pl.ds(q, _L)]
            e = jnp.exp(v)
            ebuf[pl.ds((b * _E + c) * _L, _L)] = e
            gt = v > m
            idxv = jnp.where(gt, c, idxv)
            m = jnp.maximum(m, v)
            return (s + e, m, idxv)

        s, m, idxv = pass1
        s_buf[pl.ds(b * _L, _L)] = s
        m_buf[pl.ds(b * _L, _L)] = m
        idxout[pl.ds(b * _L, _L)] = idxv
        plsc.addupdate_scatter(cnt64, [idxv], ones)
        return carry

    lax.fori_loop(0, _NBLK, block, 0)

    @plsc.parallel_loop(0, _NBLK, 1, unroll=4)
    def finalize(b):
        s = s_buf[pl.ds(b * _L, _L)]
        inv = 1.0 / s
        inv_buf[pl.ds(b * _L, _L)] = inv
        pmaxout[pl.ds(b * _L, _L)] = jnp.exp(m_buf[pl.ds(b * _L, _L)]) * inv

    def block2(b, carry):
        inv = inv_buf[pl.ds(b * _L, _L)]

        @plsc.parallel_loop(0, _E, 1, unroll=8)
        def pass2(c):
            q = ebuf[pl.ds((b * _E + c) * _L, _L)] * inv
            colacc[pl.ds(c * _L, _L)] = colacc[pl.ds(c * _L, _L)] + q

        return carry

    lax.fori_loop(0, _NBLK, block2, 0)

    for j in range(_E // _L):
        accv = zf
        for k in range(_L):
            s = jnp.sum(colacc[pl.ds((j * _L + k) * _L, _L)])
            accv = jnp.where(lane == k, jnp.broadcast_to(s, (_L,)), accv)
        partv[pl.ds(j * _L, _L)] = accv
    for j in range(_E // _L):
        partv[pl.ds(_E + j * _L, _L)] = cnt64[pl.ds(j * _L, _L)]

    pltpu.sync_copy(idxout, idx_hbm.at[pl.ds(base, _RPW)])
    pltpu.sync_copy(pmaxout, pmax_hbm.at[pl.ds(base, _RPW)])
    pltpu.sync_copy(pmaxout, pmax2_hbm.at[pl.ds(base, _RPW)])
    pltpu.sync_copy(partv, part_hbm.at[wid])


def _tc_finish_body(part_ref, *out_ref):
    p = part_ref[...]
    tot = jnp.sum(p, axis=0, keepdims=True)
    inv_b = 1.0 / _B
    mean = tot[:, :_E] * inv_b
    frac = tot[:, _E:] * inv_b
    d = jnp.sum(frac * mean)
    loss = _E * d - 1.0
    fmin = jnp.min(frac)
    fmax = jnp.max(frac)
    fmean = jnp.sum(frac) * (1.0 / _E)
    var = jnp.sum((frac - fmean) ** 2) * (1.0 / (_E - 1))
    std = jnp.sqrt(var)
    loss_ref, fmin_ref, fmax_ref, std_ref = out_ref
    loss_ref[0, 0] = loss
    fmin_ref[0, 0] = fmin
    fmax_ref[0, 0] = fmax
    std_ref[0, 0] = std


def kernel(route_logits):
    mesh = plsc.VectorSubcoreMesh(core_axis_name="c", subcore_axis_name="s")
    router = pl.kernel(
        _router_body,
        out_type=(
            jax.ShapeDtypeStruct((_B,), jnp.int32),
            jax.ShapeDtypeStruct((_B,), jnp.float32),
            jax.ShapeDtypeStruct((_B,), jnp.float32),
            jax.ShapeDtypeStruct((_NW, 2 * _E), jnp.float32),
        ),
        mesh=mesh,
        compiler_params=pltpu.CompilerParams(needs_layout_passes=False),
        scratch_types=[
            pltpu.VMEM((_E, 4, 128), jnp.float32),
            pltpu.VMEM((_RPW * _E,), jnp.float32),
            pltpu.VMEM((_E * _L,), jnp.float32),
            pltpu.VMEM((_E,), jnp.float32),
            pltpu.VMEM((_RPW,), jnp.int32),
            pltpu.VMEM((_RPW,), jnp.float32),
            pltpu.VMEM((2 * _E,), jnp.float32),
            pltpu.VMEM((_RPW,), jnp.float32),
            pltpu.VMEM((_RPW,), jnp.float32),
            pltpu.VMEM((_RPW,), jnp.float32),
            pltpu.SemaphoreType.DMA,
        ],
    )
    scalar_spec = pl.BlockSpec(memory_space=pltpu.SMEM)
    finish = pl.pallas_call(
        _tc_finish_body,
        out_shape=[jax.ShapeDtypeStruct((1, 1), jnp.float32)] * 4,
        out_specs=[scalar_spec] * 4,
    )
    xt = route_logits.T.reshape(_E, 128, 128)
    route_idx, route_prob_max, route_mult, partials = router(xt)
    loss, fmin, fmax, std = finish(partials)
    return (route_idx, route_mult, loss[0, 0], route_prob_max,
            fmin[0, 0], fmax[0, 0], std[0, 0])
